# Initial kernel scaffold; baseline (speedup 1.0000x reference)
import functools

import jax
import jax.numpy as jnp
from jax import lax
from jax.experimental import pallas as pl
from jax.experimental.pallas import tpu as pltpu
from jax.experimental.pallas import tpu_sc as plsc

N_USERS = 25000
N_NODES = 50000
N_EDGES = 800000
DIM = 64
BATCH = 4096

HALF = N_NODES // 2
ACC_ROWS = 26000
TRASH = 25008
EC = 80
SB = 10000
NSB = 5
CPS = SB // EC
EPT = N_EDGES // 16
ZCH = 125
PC = BATCH // 32

_MESH = plsc.VectorSubcoreMesh(
    core_axis_name="c", subcore_axis_name="s", num_cores=2, num_subcores=16
)

_F32 = jnp.float32
_I32 = jnp.int32


def _scale_and_localize(rows_v, ldst_v, sdst, sval, off0, base_node):
    for j in range(EC // 16):
        off = off0 + j * 16
        dvec = sdst[pl.ds(off, 16)]
        ld = dvec - base_node
        oob = (ld < 0) | (ld >= HALF)
        tv = TRASH + (dvec & 63)
        ldst_v[pl.ds(j * 16, 16)] = jnp.where(oob, tv, ld)
        for e in range(16):
            bc = plsc.load_gather(sval, [jnp.full((16,), off + e, _I32)])
            row = j * 16 + e
            for d in range(4):
                sl = rows_v[row, pl.ds(d * 16, 16)]
                rows_v[row, pl.ds(d * 16, 16)] = sl * bc


def _layer_body(ecur, srcg, dstg, valg, enext, acc, sidx, sdst, sval,
                ldst0, ldst1, rows0, rows1, stage, sem0, sem1):
    c = lax.axis_index("c")
    s = lax.axis_index("s")
    base_node = c * HALF
    zeros16 = jnp.zeros((16,), _F32)

    def _zrow(r, _):
        for d in range(4):
            stage[r, pl.ds(d * 16, 16)] = zeros16
        return 0

    lax.fori_loop(0, ZCH, _zrow, 0)
    tz = s * (ACC_ROWS // 16)

    def _zacc(i, _):
        pltpu.sync_copy(stage, acc.at[pl.ds(tz + i * ZCH, ZCH)])
        return 0

    lax.fori_loop(0, ACC_ROWS // 16 // ZCH, _zacc, 0)
    plsc.subcore_barrier()

    def _chunk(ch, rows_v, ldst_v):
        off0 = ch * EC
        _scale_and_localize(rows_v, ldst_v, sdst, sval, off0, base_node)
        pltpu.sync_copy(rows_v, acc.at[ldst_v], add=True)

    def _sb(sb, _):
        e0 = s * EPT + sb * SB
        pltpu.sync_copy(srcg.at[pl.ds(e0, SB)], sidx)
        pltpu.sync_copy(dstg.at[pl.ds(e0, SB)], sdst)
        pltpu.sync_copy(valg.at[pl.ds(e0, SB)], sval)
        pltpu.async_copy(ecur.at[sidx.at[pl.ds(0, EC)]], rows0, sem0)

        def _pair(m, _):
            ch0 = 2 * m
            pltpu.async_copy(
                ecur.at[sidx.at[pl.ds((ch0 + 1) * EC, EC)]], rows1, sem1)
            pltpu.make_async_copy(ecur.at[sidx.at[pl.ds(0, EC)]], rows0,
                                  sem0).wait()
            _chunk(ch0, rows0, ldst0)
            pltpu.async_copy(
                ecur.at[sidx.at[pl.ds((ch0 + 2) * EC, EC)]], rows0, sem0)
            pltpu.make_async_copy(ecur.at[sidx.at[pl.ds(0, EC)]], rows1,
                                  sem1).wait()
            _chunk(ch0 + 1, rows1, ldst1)
            return 0

        lax.fori_loop(0, (CPS - 1) // 2, _pair, 0)
        pltpu.make_async_copy(ecur.at[sidx.at[pl.ds(0, EC)]], rows0,
                              sem0).wait()
        _chunk(CPS - 1, rows0, ldst0)
        return 0

    lax.fori_loop(0, NSB, _sb, 0)
    plsc.subcore_barrier()

    nch = jnp.where(s < 8, 13, 12)

    def _cp(k, _):
        r0 = (s + k * 16) * ZCH
        pltpu.sync_copy(acc.at[pl.ds(r0, ZCH)],
                        enext.at[pl.ds(base_node + r0, ZCH)])
        return 0

    lax.fori_loop(0, nch, _cp, 0)


_layer = functools.partial(
    pl.kernel,
    out_type=jax.ShapeDtypeStruct((N_NODES, DIM), _F32),
    mesh=_MESH,
    scratch_types=[
        pltpu.VMEM_SHARED((ACC_ROWS, DIM), _F32),
        pltpu.VMEM((SB,), _I32),
        pltpu.VMEM((SB,), _I32),
        pltpu.VMEM((SB,), _F32),
        pltpu.VMEM((EC,), _I32),
        pltpu.VMEM((EC,), _I32),
        pltpu.VMEM((EC, DIM), _F32),
        pltpu.VMEM((EC, DIM), _F32),
        pltpu.VMEM((ZCH, DIM), _F32),
        pltpu.SemaphoreType.DMA,
        pltpu.SemaphoreType.DMA,
    ],
)(_layer_body)


def _score_body(uidx, iidx, e0, l1, l2, l3, out, uv, iv, t0, t1, t2, t3,
                usum, isum, sc_v, sem):
    c = lax.axis_index("c")
    s = lax.axis_index("s")
    wid = s * 2 + c
    base = wid * PC
    pltpu.sync_copy(uidx.at[pl.ds(base, PC)], uv)
    pltpu.sync_copy(iidx.at[pl.ds(base, PC)], iv)

    def _gather4(idx_v, dst_sum):
        pltpu.async_copy(e0.at[idx_v], t0, sem).wait()
        pltpu.async_copy(l1.at[idx_v], t1, sem).wait()
        pltpu.async_copy(l2.at[idx_v], t2, sem).wait()
        pltpu.async_copy(l3.at[idx_v], t3, sem).wait()

        def _sumr(r, _):
            for d in range(4):
                sl = pl.ds(d * 16, 16)
                dst_sum[r, sl] = (t0[r, sl] + t1[r, sl] + t2[r, sl]
                                  + t3[r, sl])
            return 0

        lax.fori_loop(0, PC, _sumr, 0)

    _gather4(uv, usum)
    _gather4(iv, isum)

    lanes = jnp.arange(16, dtype=_I32)
    for g in range(PC // 16):
        rowsel = g * 16 + lanes
        acc = jnp.zeros((16,), _F32)
        for d in range(DIM):
            col = jnp.full((16,), d, _I32)
            ua = plsc.load_gather(usum, [rowsel, col])
            ia = plsc.load_gather(isum, [rowsel, col])
            acc = acc + ua * ia
        sc_v[pl.ds(g * 16, 16)] = acc * 0.0625

    pltpu.sync_copy(sc_v, out.at[pl.ds(base, PC)])


_score = functools.partial(
    pl.kernel,
    out_type=jax.ShapeDtypeStruct((BATCH,), _F32),
    mesh=_MESH,
    scratch_types=[
        pltpu.VMEM((PC,), _I32),
        pltpu.VMEM((PC,), _I32),
        pltpu.VMEM((PC, DIM), _F32),
        pltpu.VMEM((PC, DIM), _F32),
        pltpu.VMEM((PC, DIM), _F32),
        pltpu.VMEM((PC, DIM), _F32),
        pltpu.VMEM((PC, DIM), _F32),
        pltpu.VMEM((PC, DIM), _F32),
        pltpu.VMEM((PC,), _F32),
        pltpu.SemaphoreType.DMA,
    ],
)(_score_body)


def kernel(users, items, adj_indices, adj_values, user_emb, item_emb):
    e0 = jnp.concatenate([user_emb, item_emb], axis=0)
    src = adj_indices[1].astype(_I32)
    dst = adj_indices[0].astype(_I32)
    l1 = _layer(e0, src, dst, adj_values)
    l2 = _layer(l1, src, dst, adj_values)
    l3 = _layer(l2, src, dst, adj_values)
    return _score(users.astype(_I32), (items + N_USERS).astype(_I32),
                  e0, l1, l2, l3)


# speedup vs baseline: 3.4682x; 3.4682x over previous
import functools

import jax
import jax.numpy as jnp
from jax import lax
from jax.experimental import pallas as pl
from jax.experimental.pallas import tpu as pltpu
from jax.experimental.pallas import tpu_sc as plsc

N_USERS = 25000
N_NODES = 50000
N_EDGES = 800000
DIM = 64
BATCH = 4096

HALF = N_NODES // 2
ACC_ROWS = 25600
TRASH = 25008
EC = 80
SB = 2000
NSB = 25
CPS = SB // EC
EPT = N_EDGES // 16
ZCH = 40
PC = BATCH // 32

_MESH = plsc.VectorSubcoreMesh(
    core_axis_name="c", subcore_axis_name="s", num_cores=2, num_subcores=16
)

_F32 = jnp.float32
_I32 = jnp.int32


def _scale_and_localize(rows_v, ldst_v, sdst, sval, off0, base_node):
    for j in range(EC // 16):
        off = off0 + j * 16
        dvec = sdst[pl.ds(off, 16)]
        ld = dvec - base_node
        oob = (ld < 0) | (ld >= HALF)
        tv = TRASH + (dvec & 63)
        ldst_v[pl.ds(j * 16, 16)] = jnp.where(oob, tv, ld)
        for e in range(16):
            bc = plsc.load_gather(sval, [jnp.full((16,), off + e, _I32)])
            row = j * 16 + e
            for d in range(4):
                sl = rows_v[row, pl.ds(d * 16, 16)]
                rows_v[row, pl.ds(d * 16, 16)] = sl * bc


def _layer_body(ecur, srcg, dstg, valg, enext, acc, sidx, sdst, sval,
                ldst0, ldst1, rows0, rows1, stage, sem0, sem1):
    c = lax.axis_index("c")
    s = lax.axis_index("s")
    base_node = c * HALF
    zeros16 = jnp.zeros((16,), _F32)

    def _zrow(r, _):
        for d in range(4):
            stage[r, pl.ds(d * 16, 16)] = zeros16
        return 0

    lax.fori_loop(0, ZCH, _zrow, 0)
    tz = s * (ACC_ROWS // 16)

    def _zacc(i, _):
        pltpu.sync_copy(stage, acc.at[pl.ds(tz + i * ZCH, ZCH)])
        return 0

    lax.fori_loop(0, ACC_ROWS // 16 // ZCH, _zacc, 0)
    plsc.subcore_barrier()

    def _chunk(ch, rows_v, ldst_v):
        off0 = ch * EC
        _scale_and_localize(rows_v, ldst_v, sdst, sval, off0, base_node)
        pltpu.sync_copy(rows_v, acc.at[ldst_v], add=True)

    def _sb(sb, _):
        e0 = s * EPT + sb * SB
        pltpu.sync_copy(srcg.at[pl.ds(e0, SB)], sidx)
        pltpu.sync_copy(dstg.at[pl.ds(e0, SB)], sdst)
        pltpu.sync_copy(valg.at[pl.ds(e0, SB)], sval)
        pltpu.async_copy(ecur.at[sidx.at[pl.ds(0, EC)]], rows0, sem0)

        def _pair(m, _):
            ch0 = 2 * m
            pltpu.async_copy(
                ecur.at[sidx.at[pl.ds((ch0 + 1) * EC, EC)]], rows1, sem1)
            pltpu.make_async_copy(ecur.at[sidx.at[pl.ds(0, EC)]], rows0,
                                  sem0).wait()
            _chunk(ch0, rows0, ldst0)
            pltpu.async_copy(
                ecur.at[sidx.at[pl.ds((ch0 + 2) * EC, EC)]], rows0, sem0)
            pltpu.make_async_copy(ecur.at[sidx.at[pl.ds(0, EC)]], rows1,
                                  sem1).wait()
            _chunk(ch0 + 1, rows1, ldst1)
            return 0

        lax.fori_loop(0, (CPS - 1) // 2, _pair, 0)
        pltpu.make_async_copy(ecur.at[sidx.at[pl.ds(0, EC)]], rows0,
                              sem0).wait()
        _chunk(CPS - 1, rows0, ldst0)
        return 0

    lax.fori_loop(0, NSB, _sb, 0)
    plsc.subcore_barrier()

    nch = jnp.where(s == 0, 40, 39)

    def _cp(k, _):
        r0 = (s + k * 16) * ZCH
        pltpu.sync_copy(acc.at[pl.ds(r0, ZCH)],
                        enext.at[pl.ds(base_node + r0, ZCH)])
        return 0

    lax.fori_loop(0, nch, _cp, 0)


_layer = functools.partial(
    pl.kernel,
    out_type=jax.ShapeDtypeStruct((N_NODES, DIM), _F32),
    mesh=_MESH,
    compiler_params=pltpu.CompilerParams(
        needs_layout_passes=False, use_tc_tiling_on_sc=False),
    scratch_types=[
        pltpu.VMEM_SHARED((ACC_ROWS, DIM), _F32),
        pltpu.VMEM((SB,), _I32),
        pltpu.VMEM((SB,), _I32),
        pltpu.VMEM((SB,), _F32),
        pltpu.VMEM((EC,), _I32),
        pltpu.VMEM((EC,), _I32),
        pltpu.VMEM((EC, DIM), _F32),
        pltpu.VMEM((EC, DIM), _F32),
        pltpu.VMEM((ZCH, DIM), _F32),
        pltpu.SemaphoreType.DMA,
        pltpu.SemaphoreType.DMA,
    ],
)(_layer_body)


def _score_body(uidx, iidx, e0, l1, l2, l3, out, uv, iv, t0, t1, t2, t3,
                usum, isum, sc_v, sem):
    c = lax.axis_index("c")
    s = lax.axis_index("s")
    wid = s * 2 + c
    base = wid * PC
    pltpu.sync_copy(uidx.at[pl.ds(base, PC)], uv)
    pltpu.sync_copy(iidx.at[pl.ds(base, PC)], iv)

    def _gather4(idx_v, dst_sum):
        pltpu.async_copy(e0.at[idx_v], t0, sem).wait()
        pltpu.async_copy(l1.at[idx_v], t1, sem).wait()
        pltpu.async_copy(l2.at[idx_v], t2, sem).wait()
        pltpu.async_copy(l3.at[idx_v], t3, sem).wait()

        def _sumr(r, _):
            for d in range(4):
                sl = pl.ds(d * 16, 16)
                dst_sum[r, sl] = (t0[r, sl] + t1[r, sl] + t2[r, sl]
                                  + t3[r, sl])
            return 0

        lax.fori_loop(0, PC, _sumr, 0)

    _gather4(uv, usum)
    _gather4(iv, isum)

    lanes = jnp.arange(16, dtype=_I32)
    for g in range(PC // 16):
        rowsel = g * 16 + lanes
        acc = jnp.zeros((16,), _F32)
        for d in range(DIM):
            col = jnp.full((16,), d, _I32)
            ua = plsc.load_gather(usum, [rowsel, col])
            ia = plsc.load_gather(isum, [rowsel, col])
            acc = acc + ua * ia
        sc_v[pl.ds(g * 16, 16)] = acc * 0.0625

    pltpu.sync_copy(sc_v, out.at[pl.ds(base, PC)])


_score = functools.partial(
    pl.kernel,
    out_type=jax.ShapeDtypeStruct((BATCH,), _F32),
    mesh=_MESH,
    compiler_params=pltpu.CompilerParams(
        needs_layout_passes=False, use_tc_tiling_on_sc=False),
    scratch_types=[
        pltpu.VMEM((PC,), _I32),
        pltpu.VMEM((PC,), _I32),
        pltpu.VMEM((PC, DIM), _F32),
        pltpu.VMEM((PC, DIM), _F32),
        pltpu.VMEM((PC, DIM), _F32),
        pltpu.VMEM((PC, DIM), _F32),
        pltpu.VMEM((PC, DIM), _F32),
        pltpu.VMEM((PC, DIM), _F32),
        pltpu.VMEM((PC,), _F32),
        pltpu.SemaphoreType.DMA,
    ],
)(_score_body)


def kernel(users, items, adj_indices, adj_values, user_emb, item_emb):
    e0 = jnp.concatenate([user_emb, item_emb], axis=0)
    src = adj_indices[1].astype(_I32)
    dst = adj_indices[0].astype(_I32)
    l1 = _layer(e0, src, dst, adj_values)
    l2 = _layer(l1, src, dst, adj_values)
    l3 = _layer(l2, src, dst, adj_values)
    return _score(users.astype(_I32), (items + N_USERS).astype(_I32),
                  e0, l1, l2, l3)
